# baseline (speedup 1.0000x reference)
import jax
import jax.numpy as jnp
from jax.experimental import pallas as pl
from jax.experimental.pallas import tpu as pltpu

EPS = 1e-12
BS = 1024


def _fused_ln_kernel(x_ref, pos_ref, w_ref, b_ref, o_ref):
    x = x_ref[...] + pos_ref[...][None, :, :]
    mean = jnp.mean(x, axis=-1, keepdims=True)
    xc = x - mean
    var = jnp.mean(xc * xc, axis=-1, keepdims=True)
    inv = jax.lax.rsqrt(var + EPS)
    o_ref[...] = xc * inv * w_ref[...][None, :, :] + b_ref[...][None, :, :]


def kernel(input_ids, pos_table, ln_weight, ln_bias):
    b, s, h = input_ids.shape
    w2 = ln_weight.reshape(1, h)
    b2 = ln_bias.reshape(1, h)
    grid = (s // BS, b)
    return pl.pallas_call(
        _fused_ln_kernel,
        grid=grid,
        in_specs=[
            pl.BlockSpec((1, BS, h), lambda j, i: (i, j, 0)),
            pl.BlockSpec((BS, h), lambda j, i: (j, 0)),
            pl.BlockSpec((1, h), lambda j, i: (0, 0)),
            pl.BlockSpec((1, h), lambda j, i: (0, 0)),
        ],
        out_specs=pl.BlockSpec((1, BS, h), lambda j, i: (i, j, 0)),
        out_shape=jax.ShapeDtypeStruct((b, s, h), input_ids.dtype),
        compiler_params=pltpu.CompilerParams(
            dimension_semantics=("parallel", "parallel"),
        ),
    )(input_ids, pos_table[:s], w2, b2)


# speedup vs baseline: 1.1164x; 1.1164x over previous
import jax
import jax.numpy as jnp
from jax.experimental import pallas as pl
from jax.experimental.pallas import tpu as pltpu

EPS = 1e-12
BS = 512


def _fused_ln_kernel(x_ref, pos_ref, w_ref, b_ref, o_ref):
    x = x_ref[...] + pos_ref[...][None, :, :]
    mean = jnp.mean(x, axis=-1, keepdims=True)
    xc = x - mean
    var = jnp.mean(xc * xc, axis=-1, keepdims=True)
    inv = jax.lax.rsqrt(var + EPS)
    o_ref[...] = xc * inv * w_ref[...][None, :, :] + b_ref[...][None, :, :]


def kernel(input_ids, pos_table, ln_weight, ln_bias):
    b, s, h = input_ids.shape
    w2 = ln_weight.reshape(1, h)
    b2 = ln_bias.reshape(1, h)
    grid = (s // BS,)
    return pl.pallas_call(
        _fused_ln_kernel,
        grid=grid,
        in_specs=[
            pl.BlockSpec((b, BS, h), lambda j: (0, j, 0)),
            pl.BlockSpec((BS, h), lambda j: (j, 0)),
            pl.BlockSpec((1, h), lambda j: (0, 0)),
            pl.BlockSpec((1, h), lambda j: (0, 0)),
        ],
        out_specs=pl.BlockSpec((b, BS, h), lambda j: (0, j, 0)),
        out_shape=jax.ShapeDtypeStruct((b, s, h), input_ids.dtype),
        compiler_params=pltpu.CompilerParams(
            dimension_semantics=("parallel",),
        ),
    )(input_ids, pos_table[:s], w2, b2)
